# baseline (speedup 1.0000x reference)
import functools

import jax
import jax.numpy as jnp
from jax import lax
from jax.experimental import pallas as pl
from jax.experimental.pallas import tpu as pltpu
from jax.experimental.pallas import tpu_sc as plsc

_N = 10000
_K = 16
_D = 128
_DP = _D // 2
_H = 256
_E = _N * _K


_CH = 128
_NCHUNK = _E // _CH
_NC = 2
_NS = 16
_NW = _NC * _NS


def _sc_gather_kernel(x_hbm, recv_hbm, send_hbm, rec_out, snd_out,
                      ridx, rrows, sidx, srows, rsem, ssem):
    wid = lax.axis_index("s") * _NC + lax.axis_index("c")
    nt = (_NCHUNK - wid + _NW - 1) // _NW

    def body(t, carry):
        base = (wid + t * _NW) * _CH
        pltpu.sync_copy(recv_hbm.at[pl.ds(base, _CH)], ridx)
        pltpu.sync_copy(send_hbm.at[pl.ds(base, _CH)], sidx)
        r1 = pltpu.async_copy(x_hbm.at[ridx], rrows, rsem)
        r2 = pltpu.async_copy(x_hbm.at[sidx], srows, ssem)
        r1.wait()
        r2.wait()
        pltpu.sync_copy(rrows, rec_out.at[pl.ds(base, _CH)])
        pltpu.sync_copy(srows, snd_out.at[pl.ds(base, _CH)])
        return carry

    lax.fori_loop(0, nt, body, 0)


@functools.cache
def _sc_gather():
    return pl.kernel(
        _sc_gather_kernel,
        mesh=plsc.VectorSubcoreMesh(core_axis_name="c", subcore_axis_name="s"),
        compiler_params=pltpu.CompilerParams(use_tc_tiling_on_sc=False),
        out_type=(
            jax.ShapeDtypeStruct((_E, _DP), jnp.int32),
            jax.ShapeDtypeStruct((_E, _DP), jnp.int32),
        ),
        scratch_types=[
            pltpu.VMEM((_CH,), jnp.int32),
            pltpu.VMEM((_CH, _DP), jnp.int32),
            pltpu.VMEM((_CH,), jnp.int32),
            pltpu.VMEM((_CH, _DP), jnp.int32),
            pltpu.SemaphoreType.DMA,
            pltpu.SemaphoreType.DMA,
        ],
    )


_NB = 80
_BE = _NB * _K
_GRID = _N // _NB
_NNB = 1000


def _ln(h, g, bt):
    mu = jnp.mean(h, axis=-1, keepdims=True)
    var = jnp.mean((h - mu) * (h - mu), axis=-1, keepdims=True)
    return g * ((h - mu) * lax.rsqrt(var + 1e-5)) + bt


def _unpack_bf16(p):
    bf = jnp.bfloat16
    lo = lax.bitcast_convert_type(p << 16, jnp.float32).astype(bf)
    hi = lax.bitcast_convert_type(p & jnp.int32(-65536), jnp.float32).astype(bf)
    return lo, hi


def _tc_kernel(recp, sndp, ea,
               w1re, w1ro, w1se, w1so, w1e, b1, w2, b2, w3, b3, g, bt,
               m_out, s_out):
    f32 = jnp.float32
    bf = jnp.bfloat16
    rlo, rhi = _unpack_bf16(recp[...])
    slo, shi = _unpack_bf16(sndp[...])
    h = jnp.dot(rlo, w1re[...], preferred_element_type=f32)
    h = h + jnp.dot(rhi, w1ro[...], preferred_element_type=f32)
    h = h + jnp.dot(slo, w1se[...], preferred_element_type=f32)
    h = h + jnp.dot(shi, w1so[...], preferred_element_type=f32)
    h = h + jnp.dot(ea[...].astype(bf), w1e[...], preferred_element_type=f32)
    h = jax.nn.relu(h + b1[...])
    h = jax.nn.relu(jnp.dot(h.astype(bf), w2[...], preferred_element_type=f32)
                    + b2[...])
    m = jnp.dot(h.astype(bf), w3[...], preferred_element_type=f32) + b3[...]
    mln = _ln(m, g[...], bt[...])
    m_out[...] = mln
    grp = (lax.broadcasted_iota(jnp.int32, (_D, 8), 0) // 16
           == lax.broadcasted_iota(jnp.int32, (_D, 8), 1))
    s_out[...] = jnp.dot(mln, grp.astype(f32), preferred_element_type=f32)


def _node_kernel(xin, nw1, nb1, nw2, nb2, nw3, nb3, ng, nbt, x_out):
    f32 = jnp.float32
    bf = jnp.bfloat16
    h = jax.nn.relu(
        jnp.dot(xin[...].astype(bf), nw1[...], preferred_element_type=f32)
        + nb1[...])
    h = jax.nn.relu(jnp.dot(h.astype(bf), nw2[...], preferred_element_type=f32)
                    + nb2[...])
    y = jnp.dot(h.astype(bf), nw3[...], preferred_element_type=f32) + nb3[...]
    x_out[...] = _ln(y, ng[...], nbt[...])


def _const(shape):
    return pl.BlockSpec(shape, lambda i: tuple(0 for _ in shape))


def _tc_call(recp, sndp, ea, ws, interpret=False):
    (w1re, w1ro, w1se, w1so, w1e, b1, w2, b2, w3, b3, g, bt,
     nw1, nb1, nw2, nb2, nw3, nb3, ng, nbt) = ws
    m, s = pl.pallas_call(
        _tc_kernel,
        grid=(_GRID,),
        in_specs=[
            pl.BlockSpec((_BE, _DP), lambda i: (i, 0)),
            pl.BlockSpec((_BE, _DP), lambda i: (i, 0)),
            pl.BlockSpec((_BE, _D), lambda i: (i, 0)),
            _const((_DP, _H)), _const((_DP, _H)),
            _const((_DP, _H)), _const((_DP, _H)),
            _const((_D, _H)),
            _const((1, _H)),
            _const((_H, _H)), _const((1, _H)),
            _const((_H, _D)), _const((1, _D)),
            _const((1, _D)), _const((1, _D)),
        ],
        out_specs=[
            pl.BlockSpec((_BE, _D), lambda i: (i, 0)),
            pl.BlockSpec((_BE, 8), lambda i: (i, 0)),
        ],
        out_shape=[
            jax.ShapeDtypeStruct((_E, _D), jnp.float32),
            jax.ShapeDtypeStruct((_E, 8), jnp.float32),
        ],
        interpret=interpret,
    )(recp, sndp, ea, w1re, w1ro, w1se, w1so, w1e, b1, w2, b2, w3, b3, g, bt)
    xin = s.reshape(_N, _D)
    x_out = pl.pallas_call(
        _node_kernel,
        grid=(_N // _NNB,),
        in_specs=[
            pl.BlockSpec((_NNB, _D), lambda i: (i, 0)),
            _const((_D, _H)), _const((1, _H)),
            _const((_H, _H)), _const((1, _H)),
            _const((_H, _D)), _const((1, _D)),
            _const((1, _D)), _const((1, _D)),
        ],
        out_specs=pl.BlockSpec((_NNB, _D), lambda i: (i, 0)),
        out_shape=jax.ShapeDtypeStruct((_N, _D), jnp.float32),
        interpret=interpret,
    )(xin, nw1, nb1, nw2, nb2, nw3, nb3, ng, nbt)
    return m, x_out


def _prep_weights(eW1, eb1, eW2, eb2, eW3, eb3, eg, ebt,
                  nW1, nb1, nW2, nb2, nW3, nb3, ng, nbt):
    bf = jnp.bfloat16
    w1r = eW1[:_D]
    w1s = eW1[_D:2 * _D]
    return (w1r[0::2].astype(bf), w1r[1::2].astype(bf),
            w1s[0::2].astype(bf), w1s[1::2].astype(bf),
            eW1[2 * _D:].astype(bf),
            eb1.reshape(1, _H), eW2.astype(bf), eb2.reshape(1, _H),
            eW3.astype(bf), eb3.reshape(1, _D),
            eg.reshape(1, _D), ebt.reshape(1, _D),
            nW1.astype(bf), nb1.reshape(1, _H),
            nW2.astype(bf), nb2.reshape(1, _H),
            nW3.astype(bf), nb3.reshape(1, _D),
            ng.reshape(1, _D), nbt.reshape(1, _D))


def kernel(x, edge_attr, senders, receivers, n_atoms,
           eW1, eb1, eW2, eb2, eW3, eb3, eg, ebt,
           nW1, nb1, nW2, nb2, nW3, nb3, ng, nbt):
    x_pack = lax.bitcast_convert_type(
        x.astype(jnp.bfloat16).reshape(_N, _DP, 2), jnp.int32)
    rec_p, snd_p = _sc_gather()(x_pack, receivers, senders)
    ws = _prep_weights(eW1, eb1, eW2, eb2, eW3, eb3, eg, ebt,
                       nW1, nb1, nW2, nb2, nW3, nb3, ng, nbt)
    m, x_out = _tc_call(rec_p, snd_p, edge_attr, ws)
    return (x_out, m)


# speedup vs baseline: 1.2563x; 1.2563x over previous
import functools

import jax
import jax.numpy as jnp
from jax import lax
from jax.experimental import pallas as pl
from jax.experimental.pallas import tpu as pltpu
from jax.experimental.pallas import tpu_sc as plsc

_N = 10000
_K = 16
_D = 128
_DP = _D // 2
_H = 256
_E = _N * _K


_CH = 128
_NCHUNK = _E // _CH
_NC = 2
_NS = 16
_NW = _NC * _NS


def _sc_gather_kernel(x_hbm, recv_hbm, send_hbm, rec_out, snd_out,
                      ridx, rrows, sidx, srows, rsem, ssem):
    wid = lax.axis_index("s") * _NC + lax.axis_index("c")
    nt = (_NCHUNK - wid + _NW - 1) // _NW

    def body(t, carry):
        base = (wid + t * _NW) * _CH
        pltpu.sync_copy(recv_hbm.at[pl.ds(base, _CH)], ridx)
        pltpu.sync_copy(send_hbm.at[pl.ds(base, _CH)], sidx)
        r1 = pltpu.async_copy(x_hbm.at[ridx], rrows, rsem)
        r2 = pltpu.async_copy(x_hbm.at[sidx], srows, ssem)
        r1.wait()
        r2.wait()
        pltpu.sync_copy(rrows, rec_out.at[pl.ds(base, _CH)])
        pltpu.sync_copy(srows, snd_out.at[pl.ds(base, _CH)])
        return carry

    lax.fori_loop(0, nt, body, 0)


@functools.cache
def _sc_gather():
    return pl.kernel(
        _sc_gather_kernel,
        mesh=plsc.VectorSubcoreMesh(core_axis_name="c", subcore_axis_name="s"),
        out_type=(
            jax.ShapeDtypeStruct((_E, _D), jnp.float32),
            jax.ShapeDtypeStruct((_E, _D), jnp.float32),
        ),
        scratch_types=[
            pltpu.VMEM((_CH,), jnp.int32),
            pltpu.VMEM((_CH, _D), jnp.float32),
            pltpu.VMEM((_CH,), jnp.int32),
            pltpu.VMEM((_CH, _D), jnp.float32),
            pltpu.SemaphoreType.DMA,
            pltpu.SemaphoreType.DMA,
        ],
    )


_NB = 80
_BE = _NB * _K
_GRID = _N // _NB
_NNB = 1000


def _ln(h, g, bt):
    mu = jnp.mean(h, axis=-1, keepdims=True)
    var = jnp.mean((h - mu) * (h - mu), axis=-1, keepdims=True)
    return g * ((h - mu) * lax.rsqrt(var + 1e-5)) + bt


def _tc_kernel(rec, snd, ea,
               w1r, w1s, w1e, b1, w2, b2, w3, b3, g, bt,
               m_out, s_out):
    f32 = jnp.float32
    bf = jnp.bfloat16
    h = jnp.dot(rec[...].astype(bf), w1r[...], preferred_element_type=f32)
    h = h + jnp.dot(snd[...].astype(bf), w1s[...], preferred_element_type=f32)
    h = h + jnp.dot(ea[...].astype(bf), w1e[...], preferred_element_type=f32)
    h = jax.nn.relu(h + b1[...])
    h = jax.nn.relu(jnp.dot(h.astype(bf), w2[...], preferred_element_type=f32)
                    + b2[...])
    m = jnp.dot(h.astype(bf), w3[...], preferred_element_type=f32) + b3[...]
    mln = _ln(m, g[...], bt[...])
    m_out[...] = mln
    grp = (lax.broadcasted_iota(jnp.int32, (_D, 8), 0) // 16
           == lax.broadcasted_iota(jnp.int32, (_D, 8), 1))
    s_out[...] = jnp.dot(mln, grp.astype(f32), preferred_element_type=f32)


def _node_kernel(xin, nw1, nb1, nw2, nb2, nw3, nb3, ng, nbt, x_out):
    f32 = jnp.float32
    bf = jnp.bfloat16
    h = jax.nn.relu(
        jnp.dot(xin[...].astype(bf), nw1[...], preferred_element_type=f32)
        + nb1[...])
    h = jax.nn.relu(jnp.dot(h.astype(bf), nw2[...], preferred_element_type=f32)
                    + nb2[...])
    y = jnp.dot(h.astype(bf), nw3[...], preferred_element_type=f32) + nb3[...]
    x_out[...] = _ln(y, ng[...], nbt[...])


def _const(shape):
    return pl.BlockSpec(shape, lambda i: tuple(0 for _ in shape))


def _tc_call(rec, snd, ea, ws, interpret=False):
    (w1r, w1s, w1e, b1, w2, b2, w3, b3, g, bt,
     nw1, nb1, nw2, nb2, nw3, nb3, ng, nbt) = ws
    m, s = pl.pallas_call(
        _tc_kernel,
        grid=(_GRID,),
        in_specs=[
            pl.BlockSpec((_BE, _D), lambda i: (i, 0)),
            pl.BlockSpec((_BE, _D), lambda i: (i, 0)),
            pl.BlockSpec((_BE, _D), lambda i: (i, 0)),
            _const((_D, _H)), _const((_D, _H)), _const((_D, _H)),
            _const((1, _H)),
            _const((_H, _H)), _const((1, _H)),
            _const((_H, _D)), _const((1, _D)),
            _const((1, _D)), _const((1, _D)),
        ],
        out_specs=[
            pl.BlockSpec((_BE, _D), lambda i: (i, 0)),
            pl.BlockSpec((_BE, 8), lambda i: (i, 0)),
        ],
        out_shape=[
            jax.ShapeDtypeStruct((_E, _D), jnp.float32),
            jax.ShapeDtypeStruct((_E, 8), jnp.float32),
        ],
        interpret=interpret,
    )(rec, snd, ea, w1r, w1s, w1e, b1, w2, b2, w3, b3, g, bt)
    xin = s.reshape(_N, _D)
    x_out = pl.pallas_call(
        _node_kernel,
        grid=(_N // _NNB,),
        in_specs=[
            pl.BlockSpec((_NNB, _D), lambda i: (i, 0)),
            _const((_D, _H)), _const((1, _H)),
            _const((_H, _H)), _const((1, _H)),
            _const((_H, _D)), _const((1, _D)),
            _const((1, _D)), _const((1, _D)),
        ],
        out_specs=pl.BlockSpec((_NNB, _D), lambda i: (i, 0)),
        out_shape=jax.ShapeDtypeStruct((_N, _D), jnp.float32),
        interpret=interpret,
    )(xin, nw1, nb1, nw2, nb2, nw3, nb3, ng, nbt)
    return m, x_out


def _prep_weights(eW1, eb1, eW2, eb2, eW3, eb3, eg, ebt,
                  nW1, nb1, nW2, nb2, nW3, nb3, ng, nbt):
    bf = jnp.bfloat16
    return (eW1[:_D].astype(bf), eW1[_D:2 * _D].astype(bf),
            eW1[2 * _D:].astype(bf),
            eb1.reshape(1, _H), eW2.astype(bf), eb2.reshape(1, _H),
            eW3.astype(bf), eb3.reshape(1, _D),
            eg.reshape(1, _D), ebt.reshape(1, _D),
            nW1.astype(bf), nb1.reshape(1, _H),
            nW2.astype(bf), nb2.reshape(1, _H),
            nW3.astype(bf), nb3.reshape(1, _D),
            ng.reshape(1, _D), nbt.reshape(1, _D))


def kernel(x, edge_attr, senders, receivers, n_atoms,
           eW1, eb1, eW2, eb2, eW3, eb3, eg, ebt,
           nW1, nb1, nW2, nb2, nW3, nb3, ng, nbt):
    rec_feat, snd_feat = _sc_gather()(x, receivers, senders)
    ws = _prep_weights(eW1, eb1, eW2, eb2, eW3, eb3, eg, ebt,
                       nW1, nb1, nW2, nb2, nW3, nb3, ng, nbt)
    m, x_out = _tc_call(rec_feat, snd_feat, edge_attr, ws)
    return (x_out, m)


# speedup vs baseline: 1.3682x; 1.0891x over previous
import functools

import jax
import jax.numpy as jnp
from jax import lax
from jax.experimental import pallas as pl
from jax.experimental.pallas import tpu as pltpu
from jax.experimental.pallas import tpu_sc as plsc

_N = 10000
_K = 16
_D = 128
_DP = _D // 2
_H = 256
_E = _N * _K


_CH = 128
_NCHUNK = _E // _CH
_NC = 2
_NS = 16
_NW = _NC * _NS


def _sc_gather_kernel(x_hbm, recv_hbm, send_hbm, rec_out, snd_out,
                      ridx, rrows, sidx, srows, rsem, ssem):
    wid = lax.axis_index("s") * _NC + lax.axis_index("c")
    nt = (_NCHUNK - wid + _NW - 1) // _NW

    def body(t, carry):
        base = (wid + t * _NW) * _CH
        pltpu.sync_copy(recv_hbm.at[pl.ds(base, _CH)], ridx)
        pltpu.sync_copy(send_hbm.at[pl.ds(base, _CH)], sidx)
        r1 = pltpu.async_copy(x_hbm.at[ridx], rrows, rsem)
        r2 = pltpu.async_copy(x_hbm.at[sidx], srows, ssem)
        r1.wait()
        r2.wait()
        pltpu.sync_copy(rrows, rec_out.at[pl.ds(base, _CH)])
        pltpu.sync_copy(srows, snd_out.at[pl.ds(base, _CH)])
        return carry

    lax.fori_loop(0, nt, body, 0)


@functools.cache
def _sc_gather():
    return pl.kernel(
        _sc_gather_kernel,
        mesh=plsc.VectorSubcoreMesh(core_axis_name="c", subcore_axis_name="s"),
        out_type=(
            jax.ShapeDtypeStruct((_E, _D), jnp.float32),
            jax.ShapeDtypeStruct((_E, _D), jnp.float32),
        ),
        scratch_types=[
            pltpu.VMEM((_CH,), jnp.int32),
            pltpu.VMEM((_CH, _D), jnp.float32),
            pltpu.VMEM((_CH,), jnp.int32),
            pltpu.VMEM((_CH, _D), jnp.float32),
            pltpu.SemaphoreType.DMA,
            pltpu.SemaphoreType.DMA,
        ],
    )


_NB = 200
_BE = _NB * _K
_GRID = _N // _NB
_NNB = 1000


def _ln(h, g, bt):
    mu = jnp.mean(h, axis=-1, keepdims=True)
    var = jnp.mean((h - mu) * (h - mu), axis=-1, keepdims=True)
    return g * ((h - mu) * lax.rsqrt(var + 1e-5)) + bt


def _tc_kernel(rec, snd, ea,
               w1r, w1s, w1e, b1, w2, b2, w3, b3, g, bt,
               m_out, s_out):
    f32 = jnp.float32
    bf = jnp.bfloat16
    h = jnp.dot(rec[...].astype(bf), w1r[...], preferred_element_type=f32)
    h = h + jnp.dot(snd[...].astype(bf), w1s[...], preferred_element_type=f32)
    h = h + jnp.dot(ea[...].astype(bf), w1e[...], preferred_element_type=f32)
    h = jax.nn.relu(h + b1[...])
    h = jax.nn.relu(jnp.dot(h.astype(bf), w2[...], preferred_element_type=f32)
                    + b2[...])
    m = jnp.dot(h.astype(bf), w3[...], preferred_element_type=f32) + b3[...]
    mln = _ln(m, g[...], bt[...])
    m_out[...] = mln
    grp = (lax.broadcasted_iota(jnp.int32, (_D, 8), 0) // 16
           == lax.broadcasted_iota(jnp.int32, (_D, 8), 1))
    s_out[...] = jnp.dot(mln, grp.astype(f32), preferred_element_type=f32)


def _node_kernel(xin, nw1, nb1, nw2, nb2, nw3, nb3, ng, nbt, x_out):
    f32 = jnp.float32
    bf = jnp.bfloat16
    h = jax.nn.relu(
        jnp.dot(xin[...].astype(bf), nw1[...], preferred_element_type=f32)
        + nb1[...])
    h = jax.nn.relu(jnp.dot(h.astype(bf), nw2[...], preferred_element_type=f32)
                    + nb2[...])
    y = jnp.dot(h.astype(bf), nw3[...], preferred_element_type=f32) + nb3[...]
    x_out[...] = _ln(y, ng[...], nbt[...])


def _const(shape):
    return pl.BlockSpec(shape, lambda i: tuple(0 for _ in shape))


def _tc_call(rec, snd, ea, ws, interpret=False):
    (w1r, w1s, w1e, b1, w2, b2, w3, b3, g, bt,
     nw1, nb1, nw2, nb2, nw3, nb3, ng, nbt) = ws
    m, s = pl.pallas_call(
        _tc_kernel,
        grid=(_GRID,),
        in_specs=[
            pl.BlockSpec((_BE, _D), lambda i: (i, 0)),
            pl.BlockSpec((_BE, _D), lambda i: (i, 0)),
            pl.BlockSpec((_BE, _D), lambda i: (i, 0)),
            _const((_D, _H)), _const((_D, _H)), _const((_D, _H)),
            _const((1, _H)),
            _const((_H, _H)), _const((1, _H)),
            _const((_H, _D)), _const((1, _D)),
            _const((1, _D)), _const((1, _D)),
        ],
        out_specs=[
            pl.BlockSpec((_BE, _D), lambda i: (i, 0)),
            pl.BlockSpec((_BE, 8), lambda i: (i, 0)),
        ],
        out_shape=[
            jax.ShapeDtypeStruct((_E, _D), jnp.float32),
            jax.ShapeDtypeStruct((_E, 8), jnp.float32),
        ],
        interpret=interpret,
    )(rec, snd, ea, w1r, w1s, w1e, b1, w2, b2, w3, b3, g, bt)
    xin = s.reshape(_N, _D)
    x_out = pl.pallas_call(
        _node_kernel,
        grid=(_N // _NNB,),
        in_specs=[
            pl.BlockSpec((_NNB, _D), lambda i: (i, 0)),
            _const((_D, _H)), _const((1, _H)),
            _const((_H, _H)), _const((1, _H)),
            _const((_H, _D)), _const((1, _D)),
            _const((1, _D)), _const((1, _D)),
        ],
        out_specs=pl.BlockSpec((_NNB, _D), lambda i: (i, 0)),
        out_shape=jax.ShapeDtypeStruct((_N, _D), jnp.float32),
        interpret=interpret,
    )(xin, nw1, nb1, nw2, nb2, nw3, nb3, ng, nbt)
    return m, x_out


def _prep_weights(eW1, eb1, eW2, eb2, eW3, eb3, eg, ebt,
                  nW1, nb1, nW2, nb2, nW3, nb3, ng, nbt):
    bf = jnp.bfloat16
    return (eW1[:_D].astype(bf), eW1[_D:2 * _D].astype(bf),
            eW1[2 * _D:].astype(bf),
            eb1.reshape(1, _H), eW2.astype(bf), eb2.reshape(1, _H),
            eW3.astype(bf), eb3.reshape(1, _D),
            eg.reshape(1, _D), ebt.reshape(1, _D),
            nW1.astype(bf), nb1.reshape(1, _H),
            nW2.astype(bf), nb2.reshape(1, _H),
            nW3.astype(bf), nb3.reshape(1, _D),
            ng.reshape(1, _D), nbt.reshape(1, _D))


def kernel(x, edge_attr, senders, receivers, n_atoms,
           eW1, eb1, eW2, eb2, eW3, eb3, eg, ebt,
           nW1, nb1, nW2, nb2, nW3, nb3, ng, nbt):
    rec_feat, snd_feat = _sc_gather()(x, receivers, senders)
    ws = _prep_weights(eW1, eb1, eW2, eb2, eW3, eb3, eg, ebt,
                       nW1, nb1, nW2, nb2, nW3, nb3, ng, nbt)
    m, x_out = _tc_call(rec_feat, snd_feat, edge_attr, ws)
    return (x_out, m)


# speedup vs baseline: 1.3967x; 1.0209x over previous
import functools

import jax
import jax.numpy as jnp
from jax import lax
from jax.experimental import pallas as pl
from jax.experimental.pallas import tpu as pltpu
from jax.experimental.pallas import tpu_sc as plsc

_N = 10000
_K = 16
_D = 128
_DP = _D // 2
_H = 256
_E = _N * _K


_CH = 128
_NCHUNK = _E // _CH
_NC = 2
_NS = 16
_NW = _NC * _NS


def _sc_gather_kernel(x_hbm, recv_hbm, send_hbm, rec_out, snd_out,
                      ridx, rrows, sidx, srows, rsem, ssem):
    wid = lax.axis_index("s") * _NC + lax.axis_index("c")
    nt = (_NCHUNK - wid + _NW - 1) // _NW

    def body(t, carry):
        base = (wid + t * _NW) * _CH
        pltpu.sync_copy(recv_hbm.at[pl.ds(base, _CH)], ridx)
        pltpu.sync_copy(send_hbm.at[pl.ds(base, _CH)], sidx)
        r1 = pltpu.async_copy(x_hbm.at[ridx], rrows, rsem)
        r2 = pltpu.async_copy(x_hbm.at[sidx], srows, ssem)
        r1.wait()
        r2.wait()
        pltpu.sync_copy(rrows, rec_out.at[pl.ds(base, _CH)])
        pltpu.sync_copy(srows, snd_out.at[pl.ds(base, _CH)])
        return carry

    lax.fori_loop(0, nt, body, 0)


@functools.cache
def _sc_gather():
    return pl.kernel(
        _sc_gather_kernel,
        mesh=plsc.VectorSubcoreMesh(core_axis_name="c", subcore_axis_name="s"),
        out_type=(
            jax.ShapeDtypeStruct((_E, _D), jnp.float32),
            jax.ShapeDtypeStruct((_E, _D), jnp.float32),
        ),
        scratch_types=[
            pltpu.VMEM((_CH,), jnp.int32),
            pltpu.VMEM((_CH, _D), jnp.float32),
            pltpu.VMEM((_CH,), jnp.int32),
            pltpu.VMEM((_CH, _D), jnp.float32),
            pltpu.SemaphoreType.DMA,
            pltpu.SemaphoreType.DMA,
        ],
    )


_NB = 400
_BE = _NB * _K
_GRID = _N // _NB
_NNB = 1000


def _ln(h, g, bt):
    mu = jnp.mean(h, axis=-1, keepdims=True)
    var = jnp.mean((h - mu) * (h - mu), axis=-1, keepdims=True)
    return g * ((h - mu) * lax.rsqrt(var + 1e-5)) + bt


def _tc_kernel(rec, snd, ea,
               w1r, w1s, w1e, b1, w2, b2, w3, b3, g, bt,
               m_out, s_out):
    f32 = jnp.float32
    bf = jnp.bfloat16
    h = jnp.dot(rec[...].astype(bf), w1r[...], preferred_element_type=f32)
    h = h + jnp.dot(snd[...].astype(bf), w1s[...], preferred_element_type=f32)
    h = h + jnp.dot(ea[...].astype(bf), w1e[...], preferred_element_type=f32)
    h = jax.nn.relu(h + b1[...])
    h = jax.nn.relu(jnp.dot(h.astype(bf), w2[...], preferred_element_type=f32)
                    + b2[...])
    m = jnp.dot(h.astype(bf), w3[...], preferred_element_type=f32) + b3[...]
    mln = _ln(m, g[...], bt[...])
    m_out[...] = mln
    grp = (lax.broadcasted_iota(jnp.int32, (_D, 8), 0) // 16
           == lax.broadcasted_iota(jnp.int32, (_D, 8), 1))
    s_out[...] = jnp.dot(mln, grp.astype(f32), preferred_element_type=f32)


def _node_kernel(xin, nw1, nb1, nw2, nb2, nw3, nb3, ng, nbt, x_out):
    f32 = jnp.float32
    bf = jnp.bfloat16
    h = jax.nn.relu(
        jnp.dot(xin[...].astype(bf), nw1[...], preferred_element_type=f32)
        + nb1[...])
    h = jax.nn.relu(jnp.dot(h.astype(bf), nw2[...], preferred_element_type=f32)
                    + nb2[...])
    y = jnp.dot(h.astype(bf), nw3[...], preferred_element_type=f32) + nb3[...]
    x_out[...] = _ln(y, ng[...], nbt[...])


def _const(shape):
    return pl.BlockSpec(shape, lambda i: tuple(0 for _ in shape))


def _tc_call(rec, snd, ea, ws, interpret=False):
    (w1r, w1s, w1e, b1, w2, b2, w3, b3, g, bt,
     nw1, nb1, nw2, nb2, nw3, nb3, ng, nbt) = ws
    m, s = pl.pallas_call(
        _tc_kernel,
        grid=(_GRID,),
        in_specs=[
            pl.BlockSpec((_BE, _D), lambda i: (i, 0)),
            pl.BlockSpec((_BE, _D), lambda i: (i, 0)),
            pl.BlockSpec((_BE, _D), lambda i: (i, 0)),
            _const((_D, _H)), _const((_D, _H)), _const((_D, _H)),
            _const((1, _H)),
            _const((_H, _H)), _const((1, _H)),
            _const((_H, _D)), _const((1, _D)),
            _const((1, _D)), _const((1, _D)),
        ],
        out_specs=[
            pl.BlockSpec((_BE, _D), lambda i: (i, 0)),
            pl.BlockSpec((_BE, 8), lambda i: (i, 0)),
        ],
        out_shape=[
            jax.ShapeDtypeStruct((_E, _D), jnp.float32),
            jax.ShapeDtypeStruct((_E, 8), jnp.float32),
        ],
        interpret=interpret,
    )(rec, snd, ea, w1r, w1s, w1e, b1, w2, b2, w3, b3, g, bt)
    xin = s.reshape(_N, _D)
    x_out = pl.pallas_call(
        _node_kernel,
        grid=(_N // _NNB,),
        in_specs=[
            pl.BlockSpec((_NNB, _D), lambda i: (i, 0)),
            _const((_D, _H)), _const((1, _H)),
            _const((_H, _H)), _const((1, _H)),
            _const((_H, _D)), _const((1, _D)),
            _const((1, _D)), _const((1, _D)),
        ],
        out_specs=pl.BlockSpec((_NNB, _D), lambda i: (i, 0)),
        out_shape=jax.ShapeDtypeStruct((_N, _D), jnp.float32),
        interpret=interpret,
    )(xin, nw1, nb1, nw2, nb2, nw3, nb3, ng, nbt)
    return m, x_out


def _prep_weights(eW1, eb1, eW2, eb2, eW3, eb3, eg, ebt,
                  nW1, nb1, nW2, nb2, nW3, nb3, ng, nbt):
    bf = jnp.bfloat16
    return (eW1[:_D].astype(bf), eW1[_D:2 * _D].astype(bf),
            eW1[2 * _D:].astype(bf),
            eb1.reshape(1, _H), eW2.astype(bf), eb2.reshape(1, _H),
            eW3.astype(bf), eb3.reshape(1, _D),
            eg.reshape(1, _D), ebt.reshape(1, _D),
            nW1.astype(bf), nb1.reshape(1, _H),
            nW2.astype(bf), nb2.reshape(1, _H),
            nW3.astype(bf), nb3.reshape(1, _D),
            ng.reshape(1, _D), nbt.reshape(1, _D))


def kernel(x, edge_attr, senders, receivers, n_atoms,
           eW1, eb1, eW2, eb2, eW3, eb3, eg, ebt,
           nW1, nb1, nW2, nb2, nW3, nb3, ng, nbt):
    rec_feat, snd_feat = _sc_gather()(x, receivers, senders)
    ws = _prep_weights(eW1, eb1, eW2, eb2, eW3, eb3, eg, ebt,
                       nW1, nb1, nW2, nb2, nW3, nb3, ng, nbt)
    m, x_out = _tc_call(rec_feat, snd_feat, edge_attr, ws)
    return (x_out, m)


# speedup vs baseline: 1.6614x; 1.1895x over previous
import functools

import jax
import jax.numpy as jnp
from jax import lax
from jax.experimental import pallas as pl
from jax.experimental.pallas import tpu as pltpu
from jax.experimental.pallas import tpu_sc as plsc

_N = 10000
_K = 16
_D = 128
_DP = _D // 2
_H = 256
_E = _N * _K


_P = 5
_EP = _E // _P
_CH = 128
_NCHUNK = _EP // _CH
_NC = 2
_NS = 16
_NW = _NC * _NS


def _sc_gather_kernel(x_hbm, recv_hbm, send_hbm, rec_out, snd_out,
                      ridx, rrows, sidx, srows, rsem, ssem):
    wid = lax.axis_index("s") * _NC + lax.axis_index("c")
    nt = (_NCHUNK - wid + _NW - 1) // _NW

    def body(t, carry):
        base = (wid + t * _NW) * _CH
        pltpu.sync_copy(recv_hbm.at[pl.ds(base, _CH)], ridx)
        pltpu.sync_copy(send_hbm.at[pl.ds(base, _CH)], sidx)
        r1 = pltpu.async_copy(x_hbm.at[ridx], rrows, rsem)
        r2 = pltpu.async_copy(x_hbm.at[sidx], srows, ssem)
        r1.wait()
        r2.wait()
        pltpu.sync_copy(rrows, rec_out.at[pl.ds(base, _CH)])
        pltpu.sync_copy(srows, snd_out.at[pl.ds(base, _CH)])
        return carry

    lax.fori_loop(0, nt, body, 0)


@functools.cache
def _sc_gather():
    return pl.kernel(
        _sc_gather_kernel,
        mesh=plsc.VectorSubcoreMesh(core_axis_name="c", subcore_axis_name="s"),
        out_type=(
            jax.ShapeDtypeStruct((_EP, _D), jnp.float32),
            jax.ShapeDtypeStruct((_EP, _D), jnp.float32),
        ),
        scratch_types=[
            pltpu.VMEM((_CH,), jnp.int32),
            pltpu.VMEM((_CH, _D), jnp.float32),
            pltpu.VMEM((_CH,), jnp.int32),
            pltpu.VMEM((_CH, _D), jnp.float32),
            pltpu.SemaphoreType.DMA,
            pltpu.SemaphoreType.DMA,
        ],
    )


_NB = 400
_BE = _NB * _K
_GRID = _N // _NB
_SPP = _GRID // _P
_NNB = 1000


def _ln(h, g, bt):
    mu = jnp.mean(h, axis=-1, keepdims=True)
    var = jnp.mean((h - mu) * (h - mu), axis=-1, keepdims=True)
    return g * ((h - mu) * lax.rsqrt(var + 1e-5)) + bt


def _tc_kernel(rec, snd, ea,
               w1r, w1s, w1e, b1, w2, b2, w3, b3, g, bt,
               m_out, s_out):
    f32 = jnp.float32
    bf = jnp.bfloat16
    h = jnp.dot(rec[...].astype(bf), w1r[...], preferred_element_type=f32)
    h = h + jnp.dot(snd[...].astype(bf), w1s[...], preferred_element_type=f32)
    h = h + jnp.dot(ea[...].astype(bf), w1e[...], preferred_element_type=f32)
    h = jax.nn.relu(h + b1[...])
    h = jax.nn.relu(jnp.dot(h.astype(bf), w2[...], preferred_element_type=f32)
                    + b2[...])
    m = jnp.dot(h.astype(bf), w3[...], preferred_element_type=f32) + b3[...]
    mln = _ln(m, g[...], bt[...])
    m_out[...] = mln
    grp = (lax.broadcasted_iota(jnp.int32, (_D, 8), 0) // 16
           == lax.broadcasted_iota(jnp.int32, (_D, 8), 1))
    s_out[...] = jnp.dot(mln, grp.astype(f32), preferred_element_type=f32)


def _node_kernel(xin, nw1, nb1, nw2, nb2, nw3, nb3, ng, nbt, x_out):
    f32 = jnp.float32
    bf = jnp.bfloat16
    h = jax.nn.relu(
        jnp.dot(xin[...].astype(bf), nw1[...], preferred_element_type=f32)
        + nb1[...])
    h = jax.nn.relu(jnp.dot(h.astype(bf), nw2[...], preferred_element_type=f32)
                    + nb2[...])
    y = jnp.dot(h.astype(bf), nw3[...], preferred_element_type=f32) + nb3[...]
    x_out[...] = _ln(y, ng[...], nbt[...])


def _const(shape):
    return pl.BlockSpec(shape, lambda i: tuple(0 for _ in shape))


def _tc_edge_part(part, rec_p, snd_p, ea, ws, m_prev, s_prev,
                  interpret=False):
    (w1r, w1s, w1e, b1, w2, b2, w3, b3, g, bt,
     nw1, nb1, nw2, nb2, nw3, nb3, ng, nbt) = ws
    off = part * _SPP
    part_spec = pl.BlockSpec((_BE, _D), lambda i: (i, 0))
    full_spec = pl.BlockSpec((_BE, _D), lambda i: (i + off, 0))
    in_specs = [
        part_spec, part_spec, full_spec,
        _const((_D, _H)), _const((_D, _H)), _const((_D, _H)),
        _const((1, _H)),
        _const((_H, _H)), _const((1, _H)),
        _const((_H, _D)), _const((1, _D)),
        _const((1, _D)), _const((1, _D)),
    ]
    args = [rec_p, snd_p, ea, w1r, w1s, w1e, b1, w2, b2, w3, b3, g, bt]
    aliases = {}
    if m_prev is not None:
        in_specs += [pl.BlockSpec(memory_space=pl.ANY),
                     pl.BlockSpec(memory_space=pl.ANY)]
        args += [m_prev, s_prev]
        aliases = {13: 0, 14: 1}

    def body(*refs):
        _tc_kernel(*refs[:13], refs[-2], refs[-1])

    m, s = pl.pallas_call(
        body,
        grid=(_SPP,),
        in_specs=in_specs,
        out_specs=[
            pl.BlockSpec((_BE, _D), lambda i: (i + off, 0)),
            pl.BlockSpec((_BE, 8), lambda i: (i + off, 0)),
        ],
        out_shape=[
            jax.ShapeDtypeStruct((_E, _D), jnp.float32),
            jax.ShapeDtypeStruct((_E, 8), jnp.float32),
        ],
        input_output_aliases=aliases,
        interpret=interpret,
    )(*args)
    return m, s


def _node_call(s, ws, interpret=False):
    (w1r, w1s, w1e, b1, w2, b2, w3, b3, g, bt,
     nw1, nb1, nw2, nb2, nw3, nb3, ng, nbt) = ws
    xin = s.reshape(_N, _D)
    x_out = pl.pallas_call(
        _node_kernel,
        grid=(_N // _NNB,),
        in_specs=[
            pl.BlockSpec((_NNB, _D), lambda i: (i, 0)),
            _const((_D, _H)), _const((1, _H)),
            _const((_H, _H)), _const((1, _H)),
            _const((_H, _D)), _const((1, _D)),
            _const((1, _D)), _const((1, _D)),
        ],
        out_specs=pl.BlockSpec((_NNB, _D), lambda i: (i, 0)),
        out_shape=jax.ShapeDtypeStruct((_N, _D), jnp.float32),
        interpret=interpret,
    )(xin, nw1, nb1, nw2, nb2, nw3, nb3, ng, nbt)
    return x_out


def _prep_weights(eW1, eb1, eW2, eb2, eW3, eb3, eg, ebt,
                  nW1, nb1, nW2, nb2, nW3, nb3, ng, nbt):
    bf = jnp.bfloat16
    return (eW1[:_D].astype(bf), eW1[_D:2 * _D].astype(bf),
            eW1[2 * _D:].astype(bf),
            eb1.reshape(1, _H), eW2.astype(bf), eb2.reshape(1, _H),
            eW3.astype(bf), eb3.reshape(1, _D),
            eg.reshape(1, _D), ebt.reshape(1, _D),
            nW1.astype(bf), nb1.reshape(1, _H),
            nW2.astype(bf), nb2.reshape(1, _H),
            nW3.astype(bf), nb3.reshape(1, _D),
            ng.reshape(1, _D), nbt.reshape(1, _D))


def kernel(x, edge_attr, senders, receivers, n_atoms,
           eW1, eb1, eW2, eb2, eW3, eb3, eg, ebt,
           nW1, nb1, nW2, nb2, nW3, nb3, ng, nbt):
    ws = _prep_weights(eW1, eb1, eW2, eb2, eW3, eb3, eg, ebt,
                       nW1, nb1, nW2, nb2, nW3, nb3, ng, nbt)
    gath = _sc_gather()
    m = s = None
    for p in range(_P):
        rec_p, snd_p = gath(x,
                            lax.slice(receivers, (p * _EP,), ((p + 1) * _EP,)),
                            lax.slice(senders, (p * _EP,), ((p + 1) * _EP,)))
        m, s = _tc_edge_part(p, rec_p, snd_p, edge_attr, ws, m, s)
    x_out = _node_call(s, ws)
    return (x_out, m)
